# baseline (speedup 1.0000x reference)
import functools

import jax
import jax.numpy as jnp
from jax import lax
from jax.experimental import pallas as pl
from jax.experimental.pallas import tpu as pltpu
from jax.experimental.pallas import tpu_sc as plsc

N_NODES = 10000
N_EDGES = 320000
D = 128

NP = 10240
NDUMP = 128
NROWS = NP + NDUMP
NTILES = 32
EPW = 10240
EP = NTILES * EPW
CHUNK = 128
NCHUNK = EPW // CHUNK
SUP = 40
ZROWS = NROWS // 16
OROWS = NP // 16
H = CHUNK // 2

_mesh = plsc.VectorSubcoreMesh(core_axis_name="c", subcore_axis_name="s")


@functools.partial(
    pl.kernel,
    mesh=_mesh,
    out_type=[
        jax.ShapeDtypeStruct((2 * NP,), jnp.float32),
        jax.ShapeDtypeStruct((2 * NP,), jnp.float32),
    ],
    scratch_types=[
        pltpu.VMEM((NCHUNK, CHUNK), jnp.int32),
        pltpu.VMEM((NCHUNK, CHUNK), jnp.int32),
        pltpu.VMEM((CHUNK,), jnp.float32),
        pltpu.VMEM_SHARED((NROWS,), jnp.float32),
        pltpu.VMEM_SHARED((NROWS,), jnp.float32),
        pltpu.SemaphoreType.DMA,
    ],
)
def _hist_kernel(src_hbm, dst_hbm, ones_hbm, z1_hbm,
                 dego_hbm, degi_hbm, srcb, dstb, ones_v, acco, acci, hsem):
    c = lax.axis_index("c")
    s = lax.axis_index("s")
    gid = c * 16 + s
    pltpu.sync_copy(z1_hbm.at[pl.ds(0, OROWS)], acco.at[pl.ds(s * OROWS, OROWS)])
    pltpu.sync_copy(z1_hbm.at[pl.ds(0, OROWS)], acci.at[pl.ds(s * OROWS, OROWS)])

    @pl.when(s == 0)
    def _():
        pltpu.sync_copy(z1_hbm.at[pl.ds(0, NDUMP)], acco.at[pl.ds(NP, NDUMP)])
        pltpu.sync_copy(z1_hbm.at[pl.ds(0, NDUMP)], acci.at[pl.ds(NP, NDUMP)])

    pltpu.sync_copy(src_hbm.at[pl.ds(gid * NCHUNK, NCHUNK)], srcb)
    pltpu.sync_copy(dst_hbm.at[pl.ds(gid * NCHUNK, NCHUNK)], dstb)
    pltpu.sync_copy(ones_hbm, ones_v)
    plsc.subcore_barrier()

    def issue(j, carry):
        pltpu.async_copy(ones_v, acco.at[srcb.at[j]], hsem, add=True)
        pltpu.async_copy(ones_v, acci.at[dstb.at[j]], hsem, add=True)
        return carry

    def drain(j, carry):
        pltpu.make_async_copy(ones_v, acco.at[srcb.at[0]], hsem).wait()
        pltpu.make_async_copy(ones_v, acci.at[dstb.at[0]], hsem).wait()
        return carry

    lax.fori_loop(0, NCHUNK, issue, 0)
    lax.fori_loop(0, NCHUNK, drain, 0)
    plsc.subcore_barrier()
    pltpu.sync_copy(acco.at[pl.ds(s * OROWS, OROWS)],
                    dego_hbm.at[pl.ds(c * NP + s * OROWS, OROWS)])
    pltpu.sync_copy(acci.at[pl.ds(s * OROWS, OROWS)],
                    degi_hbm.at[pl.ds(c * NP + s * OROWS, OROWS)])


@functools.partial(
    pl.kernel,
    mesh=_mesh,
    out_type=jax.ShapeDtypeStruct((2, NP, D), jnp.float32),
    scratch_types=[
        pltpu.VMEM((SUP, CHUNK), jnp.int32),
        pltpu.VMEM((SUP, CHUNK), jnp.int32),
        pltpu.VMEM((CHUNK, D), jnp.float32),
        pltpu.VMEM((CHUNK, D), jnp.float32),
        pltpu.VMEM_SHARED((NROWS, D), jnp.float32),
        pltpu.SemaphoreType.DMA,
        pltpu.SemaphoreType.DMA,
        pltpu.SemaphoreType.DMA,
        pltpu.SemaphoreType.DMA,
    ],
)
def _agg_kernel(m_hbm, src_hbm, dst_hbm, zrow_hbm, out_hbm,
                srcb, dstb, buf0, buf1, acc, gsA, gsB, ssA, ssB):
    c = lax.axis_index("c")
    s = lax.axis_index("s")
    gid = c * 16 + s
    pltpu.sync_copy(zrow_hbm, acc.at[pl.ds(s * ZROWS, ZROWS)])
    plsc.subcore_barrier()

    def gather(j, buf, sem):
        pltpu.async_copy(m_hbm.at[srcb.at[j]], buf, sem)

    def wait_g(buf, sem):
        pltpu.make_async_copy(m_hbm.at[srcb.at[0]], buf, sem).wait()

    for t in range(NCHUNK // SUP):
        pltpu.sync_copy(src_hbm.at[pl.ds(gid * NCHUNK + t * SUP, SUP)], srcb)
        pltpu.sync_copy(dst_hbm.at[pl.ds(gid * NCHUNK + t * SUP, SUP)], dstb)
        gather(0, buf0, gsA)

        def body(i, carry):
            j = 2 * i
            gather(j + 1, buf1, gsB)
            wait_g(buf0, gsA)
            pltpu.sync_copy(buf0, acc.at[dstb.at[j]], add=True)

            @pl.when(i < SUP // 2 - 1)
            def _():
                gather(j + 2, buf0, gsA)

            wait_g(buf1, gsB)
            pltpu.sync_copy(buf1, acc.at[dstb.at[j + 1]], add=True)
            return carry

        lax.fori_loop(0, SUP // 2, body, 0)
    plsc.subcore_barrier()
    pltpu.sync_copy(acc.at[pl.ds(s * OROWS, OROWS)],
                    out_hbm.at[c, pl.ds(s * OROWS, OROWS)])


def _prep_body(x_ref, do_ref, di_ref, m0_ref, ns_ref, nd_ref):
    dgo = do_ref[0] + do_ref[1]
    dgi = di_ref[0] + di_ref[1]
    ns = lax.rsqrt(jnp.maximum(dgo, 1.0))
    nd = lax.rsqrt(jnp.maximum(dgi, 1.0))
    ns_ref[...] = jnp.broadcast_to(ns[:, None], (128, 16))
    nd_ref[...] = jnp.broadcast_to(nd[:, None], (128, 16))
    m0_ref[...] = x_ref[...] * ns[:, None]


GRID = NP // 128


def _tc_prep(x_p, dego, degi):
    return pl.pallas_call(
        _prep_body,
        grid=(2 * GRID,),
        in_specs=[
            pl.BlockSpec((128, D), lambda i: (i % GRID, 0)),
            pl.BlockSpec((2, 128), lambda i: (0, i % GRID)),
            pl.BlockSpec((2, 128), lambda i: (0, i % GRID)),
        ],
        out_specs=[
            pl.BlockSpec((128, D), lambda i: (i, 0)),
            pl.BlockSpec((128, 16), lambda i: (i % GRID, 0)),
            pl.BlockSpec((128, 16), lambda i: (i % GRID, 0)),
        ],
        out_shape=[
            jax.ShapeDtypeStruct((2 * NP, D), jnp.float32),
            jax.ShapeDtypeStruct((NP, 16), jnp.float32),
            jax.ShapeDtypeStruct((NP, 16), jnp.float32),
        ],
    )(x_p, dego, degi)


def _layer_body(last, part_ref, nd_ref, ns_ref, w_ref, b_ref, out_ref):
    agg = part_ref[0] + part_ref[1]
    t = agg * nd_ref[:, 0:1]
    t = lax.dot_general(t, w_ref[...], (((1,), (0,)), ((), ())),
                        preferred_element_type=jnp.float32,
                        precision=lax.Precision.HIGHEST)
    t = jnp.maximum(t + b_ref[...], 0.0)
    if not last:
        t = t * ns_ref[:, 0:1]
    out_ref[...] = t


def _tc_layer(part, nd, ns, w, b2d, last):
    grid = GRID if last else 2 * GRID
    return pl.pallas_call(
        functools.partial(_layer_body, last),
        grid=(grid,),
        in_specs=[
            pl.BlockSpec((2, 128, D), lambda i: (0, i % GRID, 0)),
            pl.BlockSpec((128, 16), lambda i: (i % GRID, 0)),
            pl.BlockSpec((128, 16), lambda i: (i % GRID, 0)),
            pl.BlockSpec((D, D), lambda i: (0, 0)),
            pl.BlockSpec((1, D), lambda i: (0, 0)),
        ],
        out_specs=pl.BlockSpec((128, D), lambda i: (i, 0)),
        out_shape=jax.ShapeDtypeStruct(
            (NP if last else 2 * NP, D), jnp.float32),
    )(part, nd, ns, w, b2d)


def kernel(x, edge_index, W0, b0, W1, b1, W2, b2, W3, b3, W4, b4):
    src = edge_index[0].astype(jnp.int32)
    dst = edge_index[1].astype(jnp.int32)
    pad = EP - N_EDGES
    src_g = jnp.concatenate([src, jnp.zeros((pad,), jnp.int32)])
    dump = NP + (jnp.arange(pad, dtype=jnp.int32) % NDUMP)
    src_d = jnp.concatenate([src, dump]).reshape(EP // CHUNK, CHUNK)
    dst_p = jnp.concatenate([dst, dump]).reshape(EP // CHUNK, CHUNK)
    src_g = src_g.reshape(EP // CHUNK, CHUNK)
    half_off = jnp.where(jnp.arange(EP // CHUNK) >= EP // CHUNK // 2,
                         NP, 0).astype(jnp.int32)
    src_g = src_g + half_off[:, None]

    x_p = jnp.pad(x, ((0, NP - N_NODES), (0, 0)))
    zrow = jnp.zeros((ZROWS, D), jnp.float32)
    z1 = jnp.zeros((OROWS,), jnp.float32)
    ones1 = jnp.ones((CHUNK,), jnp.float32)

    dego1, degi1 = _hist_kernel(src_d, dst_p, ones1, z1)
    dego = dego1.reshape(2, NP)
    degi = degi1.reshape(2, NP)
    m, ns, nd = _tc_prep(x_p, dego, degi)

    Ws = [W0, W1, W2, W3, W4]
    bs = [b0, b1, b2, b3, b4]
    for i in range(5):
        part = _agg_kernel(m, src_g, dst_p, zrow)
        m = _tc_layer(part, nd, ns, Ws[i], bs[i].reshape(1, D), last=i == 4)
    return m[:N_NODES]


# speedup vs baseline: 1.3243x; 1.3243x over previous
import functools

import jax
import jax.numpy as jnp
from jax import lax
from jax.experimental import pallas as pl
from jax.experimental.pallas import tpu as pltpu
from jax.experimental.pallas import tpu_sc as plsc

N_NODES = 10000
N_EDGES = 320000
D = 128

NP = 10240
NDUMP = 128
NROWS = NP + NDUMP
NTILES = 32
EPW = 10240
EP = NTILES * EPW
CHUNK = 128
NCHUNK = EPW // CHUNK
SUP = 40
ZROWS = NROWS // 16
OROWS = NP // 16
H = CHUNK // 2

_mesh = plsc.VectorSubcoreMesh(core_axis_name="c", subcore_axis_name="s")


@functools.partial(
    pl.kernel,
    mesh=_mesh,
    out_type=[
        jax.ShapeDtypeStruct((2 * NP,), jnp.float32),
        jax.ShapeDtypeStruct((2 * NP,), jnp.float32),
    ],
    scratch_types=[
        pltpu.VMEM((NCHUNK, CHUNK), jnp.int32),
        pltpu.VMEM((NCHUNK, CHUNK), jnp.int32),
        pltpu.VMEM((CHUNK,), jnp.float32),
        pltpu.VMEM_SHARED((NROWS,), jnp.float32),
        pltpu.VMEM_SHARED((NROWS,), jnp.float32),
        pltpu.SemaphoreType.DMA,
    ],
)
def _hist_kernel(src_hbm, dst_hbm, ones_hbm, z1_hbm,
                 dego_hbm, degi_hbm, srcb, dstb, ones_v, acco, acci, hsem):
    c = lax.axis_index("c")
    s = lax.axis_index("s")
    gid = c * 16 + s
    pltpu.sync_copy(z1_hbm.at[pl.ds(0, OROWS)], acco.at[pl.ds(s * OROWS, OROWS)])
    pltpu.sync_copy(z1_hbm.at[pl.ds(0, OROWS)], acci.at[pl.ds(s * OROWS, OROWS)])

    @pl.when(s == 0)
    def _():
        pltpu.sync_copy(z1_hbm.at[pl.ds(0, NDUMP)], acco.at[pl.ds(NP, NDUMP)])
        pltpu.sync_copy(z1_hbm.at[pl.ds(0, NDUMP)], acci.at[pl.ds(NP, NDUMP)])

    pltpu.sync_copy(src_hbm.at[pl.ds(gid * NCHUNK, NCHUNK)], srcb)
    pltpu.sync_copy(dst_hbm.at[pl.ds(gid * NCHUNK, NCHUNK)], dstb)
    pltpu.sync_copy(ones_hbm, ones_v)
    plsc.subcore_barrier()

    def issue(j, carry):
        pltpu.async_copy(ones_v, acco.at[srcb.at[j]], hsem, add=True)
        pltpu.async_copy(ones_v, acci.at[dstb.at[j]], hsem, add=True)
        return carry

    def drain(j, carry):
        pltpu.make_async_copy(ones_v, acco.at[srcb.at[0]], hsem).wait()
        pltpu.make_async_copy(ones_v, acci.at[dstb.at[0]], hsem).wait()
        return carry

    lax.fori_loop(0, NCHUNK, issue, 0)
    lax.fori_loop(0, NCHUNK, drain, 0)
    plsc.subcore_barrier()
    pltpu.sync_copy(acco.at[pl.ds(s * OROWS, OROWS)],
                    dego_hbm.at[pl.ds(c * NP + s * OROWS, OROWS)])
    pltpu.sync_copy(acci.at[pl.ds(s * OROWS, OROWS)],
                    degi_hbm.at[pl.ds(c * NP + s * OROWS, OROWS)])


@functools.partial(
    pl.kernel,
    mesh=_mesh,
    out_type=jax.ShapeDtypeStruct((2, NP, D), jnp.float32),
    scratch_types=[
        pltpu.VMEM((SUP, CHUNK), jnp.int32),
        pltpu.VMEM((SUP, CHUNK), jnp.int32),
        pltpu.VMEM((CHUNK, D), jnp.float32),
        pltpu.VMEM((CHUNK, D), jnp.float32),
        pltpu.VMEM_SHARED((NROWS, D), jnp.float32),
        pltpu.SemaphoreType.DMA,
        pltpu.SemaphoreType.DMA,
        pltpu.SemaphoreType.DMA,
        pltpu.SemaphoreType.DMA,
    ],
)
def _agg_kernel(m_hbm, src_hbm, dst_hbm, zrow_hbm, out_hbm,
                srcb, dstb, buf0, buf1, acc, gsA, gsB, ssA, ssB):
    c = lax.axis_index("c")
    s = lax.axis_index("s")
    gid = c * 16 + s
    pltpu.sync_copy(zrow_hbm, acc.at[pl.ds(s * ZROWS, ZROWS)])
    plsc.subcore_barrier()

    def gather(j, buf, sem):
        pltpu.async_copy(m_hbm.at[srcb.at[j]], buf, sem)

    def wait_g(buf, sem):
        pltpu.make_async_copy(m_hbm.at[srcb.at[0]], buf, sem).wait()

    for t in range(NCHUNK // SUP):
        pltpu.sync_copy(src_hbm.at[pl.ds(gid * NCHUNK + t * SUP, SUP)], srcb)
        pltpu.sync_copy(dst_hbm.at[pl.ds(gid * NCHUNK + t * SUP, SUP)], dstb)
        gather(0, buf0, gsA)

        def body(i, carry):
            j = 2 * i
            gather(j + 1, buf1, gsB)
            wait_g(buf0, gsA)
            pltpu.sync_copy(buf0, acc.at[dstb.at[j]], add=True)

            @pl.when(i < SUP // 2 - 1)
            def _():
                gather(j + 2, buf0, gsA)

            wait_g(buf1, gsB)
            pltpu.sync_copy(buf1, acc.at[dstb.at[j + 1]], add=True)
            return carry

        lax.fori_loop(0, SUP // 2, body, 0)
    plsc.subcore_barrier()
    pltpu.sync_copy(acc.at[pl.ds(s * OROWS, OROWS)],
                    out_hbm.at[c, pl.ds(s * OROWS, OROWS)])


def _prep_body(x_ref, do_ref, di_ref, m0_ref, ns_ref, nd_ref):
    dgo = do_ref[0] + do_ref[1]
    dgi = di_ref[0] + di_ref[1]
    ns = lax.rsqrt(jnp.maximum(dgo, 1.0))
    nd = lax.rsqrt(jnp.maximum(dgi, 1.0))
    ns_ref[...] = jnp.broadcast_to(ns[:, None], (128, 16))
    nd_ref[...] = jnp.broadcast_to(nd[:, None], (128, 16))
    m0_ref[...] = x_ref[...] * ns[:, None]


GRID = NP // 128


def _tc_prep(x_p, dego, degi):
    return pl.pallas_call(
        _prep_body,
        grid=(GRID,),
        in_specs=[
            pl.BlockSpec((128, D), lambda i: (i, 0)),
            pl.BlockSpec((2, 128), lambda i: (0, i)),
            pl.BlockSpec((2, 128), lambda i: (0, i)),
        ],
        out_specs=[
            pl.BlockSpec((128, D), lambda i: (i, 0)),
            pl.BlockSpec((128, 16), lambda i: (i, 0)),
            pl.BlockSpec((128, 16), lambda i: (i, 0)),
        ],
        out_shape=[
            jax.ShapeDtypeStruct((NP, D), jnp.float32),
            jax.ShapeDtypeStruct((NP, 16), jnp.float32),
            jax.ShapeDtypeStruct((NP, 16), jnp.float32),
        ],
    )(x_p, dego, degi)


def _layer_body(last, part_ref, nd_ref, ns_ref, w_ref, b_ref, out_ref):
    agg = part_ref[0] + part_ref[1]
    t = agg * nd_ref[:, 0:1]
    t = lax.dot_general(t, w_ref[...], (((1,), (0,)), ((), ())),
                        preferred_element_type=jnp.float32,
                        precision=lax.Precision.HIGHEST)
    t = jnp.maximum(t + b_ref[...], 0.0)
    if not last:
        t = t * ns_ref[:, 0:1]
    out_ref[...] = t


def _tc_layer(part, nd, ns, w, b2d, last):
    return pl.pallas_call(
        functools.partial(_layer_body, last),
        grid=(GRID,),
        in_specs=[
            pl.BlockSpec((2, 128, D), lambda i: (0, i, 0)),
            pl.BlockSpec((128, 16), lambda i: (i, 0)),
            pl.BlockSpec((128, 16), lambda i: (i, 0)),
            pl.BlockSpec((D, D), lambda i: (0, 0)),
            pl.BlockSpec((1, D), lambda i: (0, 0)),
        ],
        out_specs=pl.BlockSpec((128, D), lambda i: (i, 0)),
        out_shape=jax.ShapeDtypeStruct((NP, D), jnp.float32),
    )(part, nd, ns, w, b2d)


def kernel(x, edge_index, W0, b0, W1, b1, W2, b2, W3, b3, W4, b4):
    src = edge_index[0].astype(jnp.int32)
    dst = edge_index[1].astype(jnp.int32)
    pad = EP - N_EDGES
    src_g = jnp.concatenate([src, jnp.zeros((pad,), jnp.int32)])
    dump = NP + (jnp.arange(pad, dtype=jnp.int32) % NDUMP)
    src_d = jnp.concatenate([src, dump]).reshape(EP // CHUNK, CHUNK)
    dst_p = jnp.concatenate([dst, dump]).reshape(EP // CHUNK, CHUNK)
    src_g = src_g.reshape(EP // CHUNK, CHUNK)

    x_p = jnp.pad(x, ((0, NP - N_NODES), (0, 0)))
    zrow = jnp.zeros((ZROWS, D), jnp.float32)
    z1 = jnp.zeros((OROWS,), jnp.float32)
    ones1 = jnp.ones((CHUNK,), jnp.float32)

    dego1, degi1 = _hist_kernel(src_d, dst_p, ones1, z1)
    dego = dego1.reshape(2, NP)
    degi = degi1.reshape(2, NP)
    m, ns, nd = _tc_prep(x_p, dego, degi)

    Ws = [W0, W1, W2, W3, W4]
    bs = [b0, b1, b2, b3, b4]
    for i in range(5):
        part = _agg_kernel(m, src_g, dst_p, zrow)
        m = _tc_layer(part, nd, ns, Ws[i], bs[i].reshape(1, D), last=i == 4)
    return m[:N_NODES]


# speedup vs baseline: 1.3243x; 1.0000x over previous
import functools

import jax
import jax.numpy as jnp
from jax import lax
from jax.experimental import pallas as pl
from jax.experimental.pallas import tpu as pltpu
from jax.experimental.pallas import tpu_sc as plsc

N_NODES = 10000
N_EDGES = 320000
D = 128

NP = 10240
NDUMP = 128
NROWS = NP + NDUMP
NTILES = 32
EPW = 10240
EP = NTILES * EPW
CHUNK = 128
NCHUNK = EPW // CHUNK
SUP = 40
ZROWS = NROWS // 16
OROWS = NP // 16

_mesh = plsc.VectorSubcoreMesh(core_axis_name="c", subcore_axis_name="s")


@functools.partial(
    pl.kernel,
    mesh=_mesh,
    out_type=[
        jax.ShapeDtypeStruct((2 * NP,), jnp.float32),
        jax.ShapeDtypeStruct((2 * NP,), jnp.float32),
    ],
    scratch_types=[
        pltpu.VMEM((NCHUNK, CHUNK), jnp.int32),
        pltpu.VMEM((NCHUNK, CHUNK), jnp.int32),
        pltpu.VMEM((CHUNK,), jnp.float32),
        pltpu.VMEM_SHARED((NROWS,), jnp.float32),
        pltpu.VMEM_SHARED((NROWS,), jnp.float32),
        pltpu.SemaphoreType.DMA,
    ],
)
def _hist_kernel(src_hbm, dst_hbm, ones_hbm, z1_hbm,
                 dego_hbm, degi_hbm, srcb, dstb, ones_v, acco, acci, hsem):
    c = lax.axis_index("c")
    s = lax.axis_index("s")
    gid = c * 16 + s
    pltpu.sync_copy(z1_hbm.at[pl.ds(0, OROWS)], acco.at[pl.ds(s * OROWS, OROWS)])
    pltpu.sync_copy(z1_hbm.at[pl.ds(0, OROWS)], acci.at[pl.ds(s * OROWS, OROWS)])

    @pl.when(s == 0)
    def _():
        pltpu.sync_copy(z1_hbm.at[pl.ds(0, NDUMP)], acco.at[pl.ds(NP, NDUMP)])
        pltpu.sync_copy(z1_hbm.at[pl.ds(0, NDUMP)], acci.at[pl.ds(NP, NDUMP)])

    pltpu.sync_copy(src_hbm.at[pl.ds(gid * NCHUNK, NCHUNK)], srcb)
    pltpu.sync_copy(dst_hbm.at[pl.ds(gid * NCHUNK, NCHUNK)], dstb)
    pltpu.sync_copy(ones_hbm, ones_v)
    plsc.subcore_barrier()

    def issue(j, carry):
        pltpu.async_copy(ones_v, acco.at[srcb.at[j]], hsem, add=True)
        pltpu.async_copy(ones_v, acci.at[dstb.at[j]], hsem, add=True)
        return carry

    def drain(j, carry):
        pltpu.make_async_copy(ones_v, acco.at[srcb.at[0]], hsem).wait()
        pltpu.make_async_copy(ones_v, acci.at[dstb.at[0]], hsem).wait()
        return carry

    lax.fori_loop(0, NCHUNK, issue, 0)
    lax.fori_loop(0, NCHUNK, drain, 0)
    plsc.subcore_barrier()
    pltpu.sync_copy(acco.at[pl.ds(s * OROWS, OROWS)],
                    dego_hbm.at[pl.ds(c * NP + s * OROWS, OROWS)])
    pltpu.sync_copy(acci.at[pl.ds(s * OROWS, OROWS)],
                    degi_hbm.at[pl.ds(c * NP + s * OROWS, OROWS)])


@functools.partial(
    pl.kernel,
    mesh=_mesh,
    out_type=jax.ShapeDtypeStruct((2, NP, D), jnp.float32),
    scratch_types=[
        pltpu.VMEM((SUP, CHUNK), jnp.int32),
        pltpu.VMEM((SUP, CHUNK), jnp.int32),
        pltpu.VMEM((CHUNK, D), jnp.float32),
        pltpu.VMEM((CHUNK, D), jnp.float32),
        pltpu.VMEM_SHARED((NROWS, D), jnp.float32),
        pltpu.SemaphoreType.DMA,
        pltpu.SemaphoreType.DMA,
        pltpu.SemaphoreType.DMA,
        pltpu.SemaphoreType.DMA,
    ],
)
def _agg_kernel(m_hbm, src_hbm, dst_hbm, zrow_hbm, out_hbm,
                srcb, dstb, buf0, buf1, acc, gsA, gsB, ssA, ssB):
    c = lax.axis_index("c")
    s = lax.axis_index("s")
    gid = c * 16 + s
    pltpu.sync_copy(zrow_hbm, acc.at[pl.ds(s * ZROWS, ZROWS)])
    plsc.subcore_barrier()

    def gather(j, buf, sem):
        pltpu.async_copy(m_hbm.at[srcb.at[j]], buf, sem)

    def wait_g(buf, sem):
        pltpu.make_async_copy(m_hbm.at[srcb.at[0]], buf, sem).wait()

    for t in range(NCHUNK // SUP):
        pltpu.sync_copy(src_hbm.at[pl.ds(gid * NCHUNK + t * SUP, SUP)], srcb)
        pltpu.sync_copy(dst_hbm.at[pl.ds(gid * NCHUNK + t * SUP, SUP)], dstb)
        gather(0, buf0, gsA)

        def body(i, carry):
            j = 2 * i
            gather(j + 1, buf1, gsB)
            wait_g(buf0, gsA)
            pltpu.sync_copy(buf0, acc.at[dstb.at[j]], add=True)

            @pl.when(i < SUP // 2 - 1)
            def _():
                gather(j + 2, buf0, gsA)

            wait_g(buf1, gsB)
            pltpu.sync_copy(buf1, acc.at[dstb.at[j + 1]], add=True)
            return carry

        lax.fori_loop(0, SUP // 2, body, 0)
    plsc.subcore_barrier()
    pltpu.sync_copy(acc.at[pl.ds(s * OROWS, OROWS)],
                    out_hbm.at[c, pl.ds(s * OROWS, OROWS)])


def _prep_body(x_ref, do_ref, di_ref, m0_ref, ns_ref, nd_ref):
    dgo = do_ref[0] + do_ref[1]
    dgi = di_ref[0] + di_ref[1]
    ns = lax.rsqrt(jnp.maximum(dgo, 1.0))
    nd = lax.rsqrt(jnp.maximum(dgi, 1.0))
    ns_ref[...] = jnp.broadcast_to(ns[:, None], (128, 16))
    nd_ref[...] = jnp.broadcast_to(nd[:, None], (128, 16))
    m0_ref[...] = x_ref[...] * ns[:, None]


GRID = NP // 128


def _tc_prep(x_p, dego, degi):
    return pl.pallas_call(
        _prep_body,
        grid=(GRID,),
        in_specs=[
            pl.BlockSpec((128, D), lambda i: (i, 0)),
            pl.BlockSpec((2, 128), lambda i: (0, i)),
            pl.BlockSpec((2, 128), lambda i: (0, i)),
        ],
        out_specs=[
            pl.BlockSpec((128, D), lambda i: (i, 0)),
            pl.BlockSpec((128, 16), lambda i: (i, 0)),
            pl.BlockSpec((128, 16), lambda i: (i, 0)),
        ],
        out_shape=[
            jax.ShapeDtypeStruct((NP, D), jnp.float32),
            jax.ShapeDtypeStruct((NP, 16), jnp.float32),
            jax.ShapeDtypeStruct((NP, 16), jnp.float32),
        ],
    )(x_p, dego, degi)


def _layer_body(last, part_ref, nd_ref, ns_ref, w_ref, b_ref, out_ref):
    agg = part_ref[0] + part_ref[1]
    t = agg * nd_ref[:, 0:1]
    t = lax.dot_general(t, w_ref[...], (((1,), (0,)), ((), ())),
                        preferred_element_type=jnp.float32,
                        precision=lax.Precision.HIGHEST)
    t = jnp.maximum(t + b_ref[...], 0.0)
    if not last:
        t = t * ns_ref[:, 0:1]
    out_ref[...] = t


def _tc_layer(part, nd, ns, w, b2d, last):
    return pl.pallas_call(
        functools.partial(_layer_body, last),
        grid=(GRID,),
        in_specs=[
            pl.BlockSpec((2, 128, D), lambda i: (0, i, 0)),
            pl.BlockSpec((128, 16), lambda i: (i, 0)),
            pl.BlockSpec((128, 16), lambda i: (i, 0)),
            pl.BlockSpec((D, D), lambda i: (0, 0)),
            pl.BlockSpec((1, D), lambda i: (0, 0)),
        ],
        out_specs=pl.BlockSpec((128, D), lambda i: (i, 0)),
        out_shape=jax.ShapeDtypeStruct((NP, D), jnp.float32),
    )(part, nd, ns, w, b2d)


def kernel(x, edge_index, W0, b0, W1, b1, W2, b2, W3, b3, W4, b4):
    src = edge_index[0].astype(jnp.int32)
    dst = edge_index[1].astype(jnp.int32)
    pad = EP - N_EDGES
    src_g = jnp.concatenate([src, jnp.zeros((pad,), jnp.int32)])
    dump = NP + (jnp.arange(pad, dtype=jnp.int32) % NDUMP)
    src_d = jnp.concatenate([src, dump]).reshape(EP // CHUNK, CHUNK)
    dst_p = jnp.concatenate([dst, dump]).reshape(EP // CHUNK, CHUNK)
    src_g = src_g.reshape(EP // CHUNK, CHUNK)

    x_p = jnp.pad(x, ((0, NP - N_NODES), (0, 0)))
    zrow = jnp.zeros((ZROWS, D), jnp.float32)
    z1 = jnp.zeros((OROWS,), jnp.float32)
    ones1 = jnp.ones((CHUNK,), jnp.float32)

    dego1, degi1 = _hist_kernel(src_d, dst_p, ones1, z1)
    dego = dego1.reshape(2, NP)
    degi = degi1.reshape(2, NP)
    m, ns, nd = _tc_prep(x_p, dego, degi)

    Ws = [W0, W1, W2, W3, W4]
    bs = [b0, b1, b2, b3, b4]
    for i in range(5):
        part = _agg_kernel(m, src_g, dst_p, zrow)
        m = _tc_layer(part, nd, ns, Ws[i], bs[i].reshape(1, D), last=i == 4)
    return m[:N_NODES]
